# baseline (speedup 1.0000x reference)
import functools

import numpy as np
import jax
import jax.numpy as jnp
from jax.experimental import pallas as pl
from jax.experimental.pallas import tpu as pltpu

_VMEM_LIMIT = 32 * 1024 * 1024
_BLK = 128
_KMAX = 3


def _lrelu(x):
    return jnp.where(x >= 0.0, x, 0.2 * x)


def _mlp_kernel(nb_ref, x6_ref, wlc_ref, blc_ref, wlf_ref, blf_ref,
                w1_ref, b1_ref, w2a_ref, w2b_ref, b2_ref, w3_ref, b3_ref,
                o_ref):
    f32 = jnp.float32
    bf = jnp.bfloat16
    h = _lrelu(jnp.dot(nb_ref[...], wlc_ref[...], preferred_element_type=f32)
               + blc_ref[...])
    h = _lrelu(jnp.dot(h.astype(bf), wlf_ref[...], preferred_element_type=f32)
               + blf_ref[...])
    p = _lrelu(jnp.dot(x6_ref[...], w1_ref[...], preferred_element_type=f32)
               + b1_ref[...])
    q = _lrelu(jnp.dot(p.astype(bf), w2a_ref[...], preferred_element_type=f32)
               + jnp.dot(h.astype(bf), w2b_ref[...], preferred_element_type=f32)
               + b2_ref[...])
    o_ref[...] = _lrelu(
        jnp.dot(q.astype(bf), w3_ref[...], preferred_element_type=f32)
        + b3_ref[...]).astype(o_ref.dtype)


def _mlp_chain(nb_p, x6_p, wlc, blc, wlf, blf, w1, b1, w2a, w2b, b2, w3, b3,
               tm=512):
    Mp, Knb = nb_p.shape
    K6 = x6_p.shape[1]
    Dout = w3.shape[1]

    def full(a):
        return pl.BlockSpec(a.shape, lambda i, nd=a.ndim: (0,) * nd)

    return pl.pallas_call(
        _mlp_kernel,
        out_shape=jax.ShapeDtypeStruct((Mp, Dout), jnp.bfloat16),
        grid=(Mp // tm,),
        in_specs=[
            pl.BlockSpec((tm, Knb), lambda i: (i, 0)),
            pl.BlockSpec((tm, K6), lambda i: (i, 0)),
            full(wlc), full(blc), full(wlf), full(blf),
            full(w1), full(b1), full(w2a), full(w2b), full(b2),
            full(w3), full(b3),
        ],
        out_specs=pl.BlockSpec((tm, Dout), lambda i: (i, 0)),
        compiler_params=pltpu.CompilerParams(
            dimension_semantics=("parallel",),
            disable_bounds_checks=True,
            vmem_limit_bytes=_VMEM_LIMIT),
    )(nb_p, x6_p, wlc, blc, wlf, blf, w1, b1, w2a, w2b, b2, w3, b3)


def _gcn_kernel(ss_ref, cnt_ref, cols_ref,
                x0_ref, x1_ref, x2_ref, a0_ref, a1_ref, a2_ref,
                w_ref, b_ref, o_ref, *, slope, final_cfg):
    del ss_ref, cols_ref
    f32 = jnp.float32
    i = pl.program_id(0)
    cnt = cnt_ref[i]

    acc = jnp.dot(a0_ref[0], x0_ref[...], preferred_element_type=f32)
    d1 = jnp.dot(a1_ref[0], x1_ref[...], preferred_element_type=f32)
    acc = acc + jnp.where(cnt >= 2, d1, 0.0)
    d2 = jnp.dot(a2_ref[0], x2_ref[...], preferred_element_type=f32)
    acc = acc + jnp.where(cnt >= 3, d2, 0.0)

    y = jnp.dot(acc.astype(jnp.bfloat16), w_ref[...],
                preferred_element_type=f32) + b_ref[...]
    if final_cfg is None:
        y = jnp.where(y >= 0.0, y, slope * y)
    else:
        sf, ncls = final_cfg
        col = jax.lax.broadcasted_iota(jnp.int32, y.shape, 1)
        is_cls = (col >= 3) & (col < 3 + ncls)
        mx = jnp.max(jnp.where(is_cls, y, -1e30), axis=-1, keepdims=True)
        ssum = jnp.sum(jnp.where(is_cls, jnp.exp(y - mx), 0.0),
                       axis=-1, keepdims=True)
        logsm = (y - mx) - jnp.log(ssum)
        y = jnp.where(col < 3, y * sf, jnp.where(is_cls, logsm, 0.0))
    o_ref[...] = y.astype(o_ref.dtype)


def _gcn_banded(seg_start, seg_cnt, cols, a_blocks, x, w, b, *,
                slope=0.2, final_cfg=None, out_dtype=jnp.bfloat16):
    T = a_blocks.shape[0]
    Mp, Kin = x.shape
    Np = w.shape[1]
    nbk = Mp // _BLK
    x = x.astype(jnp.bfloat16)
    w = w.astype(jnp.bfloat16)

    def xmap(k):
        def f(i, ss, cnt, cc):
            return (cc[jnp.minimum(ss[i] + k, T - 1)], 0)
        return f

    def amap(k):
        def f(i, ss, cnt, cc):
            return (jnp.minimum(ss[i] + k, T - 1), 0, 0)
        return f

    grid_spec = pltpu.PrefetchScalarGridSpec(
        num_scalar_prefetch=3,
        grid=(nbk,),
        in_specs=[
            pl.BlockSpec((_BLK, Kin), xmap(0)),
            pl.BlockSpec((_BLK, Kin), xmap(1)),
            pl.BlockSpec((_BLK, Kin), xmap(2)),
            pl.BlockSpec((1, _BLK, _BLK), amap(0)),
            pl.BlockSpec((1, _BLK, _BLK), amap(1)),
            pl.BlockSpec((1, _BLK, _BLK), amap(2)),
            pl.BlockSpec((Kin, Np), lambda i, *_: (0, 0)),
            pl.BlockSpec((1, Np), lambda i, *_: (0, 0)),
        ],
        out_specs=pl.BlockSpec((_BLK, Np), lambda i, *_: (i, 0)),
    )
    body = functools.partial(_gcn_kernel, slope=slope, final_cfg=final_cfg)
    return pl.pallas_call(
        body,
        out_shape=jax.ShapeDtypeStruct((Mp, Np), out_dtype),
        grid_spec=grid_spec,
        compiler_params=pltpu.CompilerParams(
            dimension_semantics=("parallel",),
            disable_bounds_checks=True,
            vmem_limit_bytes=_VMEM_LIMIT),
    )(seg_start, seg_cnt, cols, x, x, x, a_blocks, a_blocks, a_blocks, w, b)


def _interp_kernel(g_ref, meta_ref, o_ref, *, doff, dmax):
    f32 = jnp.float32
    meta = meta_ref[...]
    offs = [float(d) for d in doff]

    def taps(p, b):
        t = jnp.concatenate([jnp.clip(p + d, 0.0, dmax) - b for d in offs],
                            axis=0)
        k0 = jnp.floor(t)
        fr = t - k0
        k0 = k0.astype(jnp.int32)[:, None, :]
        kio = jax.lax.broadcasted_iota(jnp.int32, (5, 7, 128), 1)
        return (jnp.where(kio == k0, (1.0 - fr)[:, None, :], 0.0)
                + jnp.where(kio == k0 + 1, fr[:, None, :], 0.0))

    wx = taps(meta[0:1], meta[3:4])
    wy = taps(meta[1:2], meta[4:5])
    wz = taps(meta[2:3], meta[5:6])

    B = g_ref[...]
    c1 = jnp.stack([jnp.sum(B * wx[a][None, None], axis=2)
                    for a in range(5)])
    c2 = jnp.stack([jnp.sum(c1 * wy[b][None, None], axis=2)
                    for b in range(5)])
    rows = []
    for a in range(5):
        for b in range(5):
            for c in range(5):
                rows.append(jnp.sum(c2[b, a] * wz[c], axis=0, keepdims=True))
    rows.append(jnp.zeros((3, 128), f32))
    nbt = jnp.concatenate(rows, axis=0)
    o_ref[...] = nbt.T.astype(o_ref.dtype)


def _interp_cubes(g7, meta, doff, D, tv=128):
    m = g7.shape[-1]
    body = functools.partial(_interp_kernel, doff=doff, dmax=float(D - 1))
    return pl.pallas_call(
        body,
        out_shape=jax.ShapeDtypeStruct((m, 128), jnp.bfloat16),
        grid=(m // tv,),
        in_specs=[
            pl.BlockSpec((7, 7, 7, tv), lambda t: (0, 0, 0, t)),
            pl.BlockSpec((8, tv), lambda t: (0, t)),
        ],
        out_specs=pl.BlockSpec((tv, 128), lambda t: (t, 0)),
        compiler_params=pltpu.CompilerParams(
            dimension_semantics=("parallel",),
            disable_bounds_checks=True,
            vmem_limit_bytes=_VMEM_LIMIT),
    )(g7.astype(jnp.float32), meta.astype(jnp.float32))


def _cube_shift(K):
    g = np.linspace(-K // 2, K // 2, K)
    g3 = np.stack(np.meshgrid(g, g, g), axis=0).transpose(2, 1, 3, 0)
    return jnp.asarray(g3.reshape(-1, 3), jnp.float32)


def _trilinear_border(vol, pts):
    D1, D2, D3 = vol.shape

    def pix(c, size):
        return jnp.clip((c + 1.0) * 0.5 * (size - 1), 0.0, size - 1.0)

    px = pix(pts[:, 0], D3)
    py = pix(pts[:, 1], D2)
    pz = pix(pts[:, 2], D1)
    x0f, y0f, z0f = jnp.floor(px), jnp.floor(py), jnp.floor(pz)
    wx, wy, wz = px - x0f, py - y0f, pz - z0f

    def ii(fv, size):
        return jnp.clip(fv, 0, size - 1).astype(jnp.int32)

    x0, x1 = ii(x0f, D3), ii(x0f + 1, D3)
    y0, y1 = ii(y0f, D2), ii(y0f + 1, D2)
    z0, z1 = ii(z0f, D1), ii(z0f + 1, D1)

    def g(zi, yi, xi):
        return vol[zi, yi, xi]

    c00 = g(z0, y0, x0) * (1 - wx) + g(z0, y0, x1) * wx
    c01 = g(z0, y1, x0) * (1 - wx) + g(z0, y1, x1) * wx
    c10 = g(z1, y0, x0) * (1 - wx) + g(z1, y0, x1) * wx
    c11 = g(z1, y1, x0) * (1 - wx) + g(z1, y1, x1) * wx
    c0 = c00 * (1 - wy) + c01 * wy
    c1 = c10 * (1 - wy) + c11 * wy
    return c0 * (1 - wz) + c1 * wz


def _scatter_kernel(i0_ref, i1_ref, i2_ref, nf_ref, o_ref, *, bs, nbi):
    g = pl.program_id(0)
    nb = pl.program_id(1)

    @pl.when(nb == 0)
    def _():
        o_ref[...] = jnp.zeros_like(o_ref)

    base = (g * nbi + nb) * bs

    def body(j, carry):
        a = i0_ref[base + j]
        b = i1_ref[base + j]
        c = i2_ref[base + j]
        sa = o_ref[pl.ds(a, 1), 0, :] + nf_ref[pl.ds(3 * j, 1), 0, :]
        sb = o_ref[pl.ds(b, 1), 0, :] + nf_ref[pl.ds(3 * j + 1, 1), 0, :]
        sc = o_ref[pl.ds(c, 1), 0, :] + nf_ref[pl.ds(3 * j + 2, 1), 0, :]
        o_ref[pl.ds(a, 1), 0, :] = sa
        o_ref[pl.ds(b, 1), 0, :] = sb
        o_ref[pl.ds(c, 1), 0, :] = sc
        return carry

    jax.lax.fori_loop(0, bs, body, 0)


def _vertex_normals(v, faces, m):
    i0, i1, i2 = faces[:, 0], faces[:, 1], faces[:, 2]
    nf = jnp.cross(v[i1] - v[i0], v[i2] - v[i0])
    F = nf.shape[0]
    c10 = i1 == i0
    c20 = i2 == i0
    c21 = i2 == i1
    f1 = lambda c: c.astype(jnp.float32)
    w0 = 1.0 + f1(c10) + f1(c20)
    w1 = (1.0 - f1(c10)) * (1.0 + f1(c21) * (1.0 - f1(c20)))
    w2 = f1(~(c20 | c21))
    i1 = jnp.where(c10, m, i1)
    i2 = jnp.where(c20 | c21, m + 1, i2)
    ups = jnp.stack([nf * w0[:, None], nf * w1[:, None], nf * w2[:, None]],
                    axis=1).reshape(3 * F, 1, 3)
    BS = 4096
    NBI = (F // 2) // BS
    out = pl.pallas_call(
        functools.partial(_scatter_kernel, bs=BS, nbi=NBI),
        out_shape=jax.ShapeDtypeStruct((2 * (m + 8), 1, 3), jnp.float32),
        grid_spec=pltpu.PrefetchScalarGridSpec(
            num_scalar_prefetch=3,
            grid=(2, NBI),
            in_specs=[pl.BlockSpec((3 * BS, 1, 3),
                                   lambda g, nb, *pf: (g * NBI + nb, 0, 0))],
            out_specs=pl.BlockSpec((m + 8, 1, 3), lambda g, nb, *pf: (g, 0, 0)),
        ),
        compiler_params=pltpu.CompilerParams(
            dimension_semantics=("parallel", "arbitrary"),
            disable_bounds_checks=True,
            vmem_limit_bytes=48 * 1024 * 1024),
    )(i0, i1, i2, ups)
    n = out.reshape(2, m + 8, 3)[0, :m] + out.reshape(2, m + 8, 3)[1, :m]
    return n / jnp.maximum(jnp.linalg.norm(n, axis=1, keepdims=True), 1e-6)


def _pad_cols(a, Np, dtype=jnp.bfloat16):
    m, n = a.shape
    return jnp.zeros((m, Np), dtype).at[:, :n].set(a.astype(dtype))


def kernel(x, V, f, blk_rows, blk_cols, blk_firsts, blk_lasts, A_blocks,
           wlc, blc, wlf, blf, w1, b1, w2a, w2b, b2, w3, b3,
           gcn_w0, gcn_b0, gcn_w1, gcn_b1, gcn_w2, gcn_b2,
           gcn_w3, gcn_b3, gcn_w4, gcn_b4):
    del blk_firsts, blk_lasts
    K, sf, ncls = 5, 0.1, 10
    m = x.shape[1]
    vol = V[0, 0]
    D1, D2, D3 = vol.shape
    D = max(D1, D2, D3)
    v = x[0]

    shift = _cube_shift(K) * (2.0 / D)
    rescale = jnp.asarray([D3 / D, D2 / D, D1 / D], jnp.float32)
    del rescale, shift
    doff = np.linspace(-K // 2, K // 2, K) * (2.0 / D) * 0.5 * (D - 1)

    def base_of(c):
        pix = (c + 1.0) * 0.5 * (D - 1)
        smin = jnp.clip(pix + doff[0], 0.0, D - 1.0)
        return jnp.clip(jnp.floor(smin), 0, D - 7).astype(jnp.int32), pix

    bx, px = base_of(v[:, 0])
    by, py = base_of(v[:, 1])
    bz, pz = base_of(v[:, 2])
    ii = np.arange(7, dtype=np.int32)
    off3 = ((ii[:, None, None] * D + ii[None, :, None]) * D
            + ii[None, None, :])
    fbase = (bz * D + by) * D + bx
    idx = jnp.asarray(off3)[..., None] + fbase[None, None, None, :]
    g7 = vol.reshape(-1).at[idx].get(
        mode="promise_in_bounds")
    meta = jnp.stack([px, py, pz, bx.astype(jnp.float32),
                      by.astype(jnp.float32), bz.astype(jnp.float32),
                      jnp.zeros_like(px), jnp.zeros_like(px)])
    nb_p = _interp_cubes(g7, meta, doff, D)

    fc = f[0]
    vv = v.at[fc.reshape(-1)].get(mode="promise_in_bounds").reshape(-1, 3, 3)
    nfc = jnp.cross(vv[:, 1] - vv[:, 0], vv[:, 2] - vv[:, 0])
    nsum = jnp.zeros_like(v).at[fc.T.reshape(-1)].add(
        jnp.broadcast_to(nfc, (3,) + nfc.shape).reshape(-1, 3),
        mode="promise_in_bounds")
    normal = nsum / jnp.maximum(
        jnp.linalg.norm(nsum, axis=1, keepdims=True), 1e-6)
    x6_p = _pad_cols(jnp.concatenate([v, normal], axis=1), 128)
    z = _mlp_chain(nb_p, x6_p, wlc, blc, wlf, blf,
                   w1, b1, w2a, w2b, b2, w3, b3)

    nbk = m // _BLK
    rb = jnp.arange(nbk, dtype=jnp.int32)
    seg_start = jnp.searchsorted(blk_rows, rb, side="left").astype(jnp.int32)
    seg_end = jnp.searchsorted(blk_rows, rb, side="right").astype(jnp.int32)
    seg_cnt = seg_end - seg_start

    h = z
    for (wg, bg) in ((gcn_w0, gcn_b0), (gcn_w1, gcn_b1),
                     (gcn_w2, gcn_b2), (gcn_w3, gcn_b3)):
        h = _gcn_banded(seg_start, seg_cnt, blk_cols, A_blocks, h, wg, bg,
                        slope=0.2, out_dtype=jnp.bfloat16)
    out = _gcn_banded(seg_start, seg_cnt, blk_cols, A_blocks, h,
                      gcn_w4, gcn_b4, final_cfg=(sf, ncls),
                      out_dtype=jnp.float32)

    dx = out[:m, :3]
    logits = out[:m, 3:3 + ncls]
    return dx[None], logits


# speedup vs baseline: 1.0817x; 1.0817x over previous
import functools

import numpy as np
import jax
import jax.numpy as jnp
from jax.experimental import pallas as pl
from jax.experimental.pallas import tpu as pltpu

_VMEM_LIMIT = 32 * 1024 * 1024
_BLK = 128
_KMAX = 3


def _lrelu(x):
    return jnp.where(x >= 0.0, x, 0.2 * x)


def _mlp_kernel(nb_ref, x6_ref, wlc_ref, blc_ref, wlf_ref, blf_ref,
                w1_ref, b1_ref, w2a_ref, w2b_ref, b2_ref, w3_ref, b3_ref,
                o_ref):
    f32 = jnp.float32
    bf = jnp.bfloat16
    h = _lrelu(jnp.dot(nb_ref[...], wlc_ref[...], preferred_element_type=f32)
               + blc_ref[...])
    h = _lrelu(jnp.dot(h.astype(bf), wlf_ref[...], preferred_element_type=f32)
               + blf_ref[...])
    p = _lrelu(jnp.dot(x6_ref[...], w1_ref[...], preferred_element_type=f32)
               + b1_ref[...])
    q = _lrelu(jnp.dot(p.astype(bf), w2a_ref[...], preferred_element_type=f32)
               + jnp.dot(h.astype(bf), w2b_ref[...], preferred_element_type=f32)
               + b2_ref[...])
    o_ref[...] = _lrelu(
        jnp.dot(q.astype(bf), w3_ref[...], preferred_element_type=f32)
        + b3_ref[...]).astype(o_ref.dtype)


def _mlp_chain(nb_p, x6_p, wlc, blc, wlf, blf, w1, b1, w2a, w2b, b2, w3, b3,
               tm=512):
    Mp, Knb = nb_p.shape
    K6 = x6_p.shape[1]
    Dout = w3.shape[1]

    def full(a):
        return pl.BlockSpec(a.shape, lambda i, nd=a.ndim: (0,) * nd)

    return pl.pallas_call(
        _mlp_kernel,
        out_shape=jax.ShapeDtypeStruct((Mp, Dout), jnp.bfloat16),
        grid=(Mp // tm,),
        in_specs=[
            pl.BlockSpec((tm, Knb), lambda i: (i, 0)),
            pl.BlockSpec((tm, K6), lambda i: (i, 0)),
            full(wlc), full(blc), full(wlf), full(blf),
            full(w1), full(b1), full(w2a), full(w2b), full(b2),
            full(w3), full(b3),
        ],
        out_specs=pl.BlockSpec((tm, Dout), lambda i: (i, 0)),
        compiler_params=pltpu.CompilerParams(
            dimension_semantics=("parallel",),
            disable_bounds_checks=True,
            vmem_limit_bytes=_VMEM_LIMIT),
    )(nb_p, x6_p, wlc, blc, wlf, blf, w1, b1, w2a, w2b, b2, w3, b3)


def _gcn_kernel(ss_ref, cnt_ref, cols_ref,
                x0_ref, x1_ref, x2_ref, a0_ref, a1_ref, a2_ref,
                w_ref, b_ref, o_ref, *, slope, final_cfg):
    del ss_ref, cols_ref
    f32 = jnp.float32
    i = pl.program_id(0)
    cnt = cnt_ref[i]

    acc = jnp.dot(a0_ref[0], x0_ref[...], preferred_element_type=f32)
    d1 = jnp.dot(a1_ref[0], x1_ref[...], preferred_element_type=f32)
    acc = acc + jnp.where(cnt >= 2, d1, 0.0)
    d2 = jnp.dot(a2_ref[0], x2_ref[...], preferred_element_type=f32)
    acc = acc + jnp.where(cnt >= 3, d2, 0.0)

    y = jnp.dot(acc.astype(jnp.bfloat16), w_ref[...],
                preferred_element_type=f32) + b_ref[...]
    if final_cfg is None:
        y = jnp.where(y >= 0.0, y, slope * y)
    else:
        sf, ncls = final_cfg
        col = jax.lax.broadcasted_iota(jnp.int32, y.shape, 1)
        is_cls = (col >= 3) & (col < 3 + ncls)
        mx = jnp.max(jnp.where(is_cls, y, -1e30), axis=-1, keepdims=True)
        ssum = jnp.sum(jnp.where(is_cls, jnp.exp(y - mx), 0.0),
                       axis=-1, keepdims=True)
        logsm = (y - mx) - jnp.log(ssum)
        y = jnp.where(col < 3, y * sf, jnp.where(is_cls, logsm, 0.0))
    o_ref[...] = y.astype(o_ref.dtype)


def _gcn_banded(seg_start, seg_cnt, cols, a_blocks, x, w, b, *,
                slope=0.2, final_cfg=None, out_dtype=jnp.bfloat16):
    T = a_blocks.shape[0]
    Mp, Kin = x.shape
    Np = w.shape[1]
    nbk = Mp // _BLK
    x = x.astype(jnp.bfloat16)
    w = w.astype(jnp.bfloat16)

    def xmap(k):
        def f(i, ss, cnt, cc):
            return (cc[jnp.minimum(ss[i] + k, T - 1)], 0)
        return f

    def amap(k):
        def f(i, ss, cnt, cc):
            return (jnp.minimum(ss[i] + k, T - 1), 0, 0)
        return f

    grid_spec = pltpu.PrefetchScalarGridSpec(
        num_scalar_prefetch=3,
        grid=(nbk,),
        in_specs=[
            pl.BlockSpec((_BLK, Kin), xmap(0)),
            pl.BlockSpec((_BLK, Kin), xmap(1)),
            pl.BlockSpec((_BLK, Kin), xmap(2)),
            pl.BlockSpec((1, _BLK, _BLK), amap(0)),
            pl.BlockSpec((1, _BLK, _BLK), amap(1)),
            pl.BlockSpec((1, _BLK, _BLK), amap(2)),
            pl.BlockSpec((Kin, Np), lambda i, *_: (0, 0)),
            pl.BlockSpec((1, Np), lambda i, *_: (0, 0)),
        ],
        out_specs=pl.BlockSpec((_BLK, Np), lambda i, *_: (i, 0)),
    )
    body = functools.partial(_gcn_kernel, slope=slope, final_cfg=final_cfg)
    return pl.pallas_call(
        body,
        out_shape=jax.ShapeDtypeStruct((Mp, Np), out_dtype),
        grid_spec=grid_spec,
        compiler_params=pltpu.CompilerParams(
            dimension_semantics=("parallel",),
            disable_bounds_checks=True,
            vmem_limit_bytes=_VMEM_LIMIT),
    )(seg_start, seg_cnt, cols, x, x, x, a_blocks, a_blocks, a_blocks, w, b)


def _interp_kernel(g_ref, meta_ref, o_ref, *, doff, dmax):
    f32 = jnp.float32
    meta = meta_ref[...]
    offs = [float(d) for d in doff]

    def taps(p, b):
        t = jnp.concatenate([jnp.clip(p + d, 0.0, dmax) - b for d in offs],
                            axis=0)
        k0 = jnp.floor(t)
        fr = t - k0
        k0 = k0.astype(jnp.int32)[:, None, :]
        kio = jax.lax.broadcasted_iota(jnp.int32, (5, 7, 128), 1)
        return (jnp.where(kio == k0, (1.0 - fr)[:, None, :], 0.0)
                + jnp.where(kio == k0 + 1, fr[:, None, :], 0.0))

    wx = taps(meta[0:1], meta[3:4])
    wy = taps(meta[1:2], meta[4:5])
    wz = taps(meta[2:3], meta[5:6])

    B = g_ref[...]
    c1 = jnp.stack([jnp.sum(B * wx[a][None, None], axis=2)
                    for a in range(5)])
    c2 = jnp.stack([jnp.sum(c1 * wy[b][None, None], axis=2)
                    for b in range(5)])
    rows = []
    for a in range(5):
        for b in range(5):
            for c in range(5):
                rows.append(jnp.sum(c2[b, a] * wz[c], axis=0, keepdims=True))
    rows.append(jnp.zeros((3, 128), f32))
    nbt = jnp.concatenate(rows, axis=0)
    o_ref[...] = nbt.T.astype(o_ref.dtype)


def _interp_cubes(g7, meta, doff, D, tv=128):
    m = g7.shape[-1]
    body = functools.partial(_interp_kernel, doff=doff, dmax=float(D - 1))
    return pl.pallas_call(
        body,
        out_shape=jax.ShapeDtypeStruct((m, 128), jnp.bfloat16),
        grid=(m // tv,),
        in_specs=[
            pl.BlockSpec((7, 7, 7, tv), lambda t: (0, 0, 0, t)),
            pl.BlockSpec((8, tv), lambda t: (0, t)),
        ],
        out_specs=pl.BlockSpec((tv, 128), lambda t: (t, 0)),
        compiler_params=pltpu.CompilerParams(
            dimension_semantics=("parallel",),
            disable_bounds_checks=True,
            vmem_limit_bytes=_VMEM_LIMIT),
    )(g7.astype(jnp.float32), meta.astype(jnp.float32))


def _cube_shift(K):
    g = np.linspace(-K // 2, K // 2, K)
    g3 = np.stack(np.meshgrid(g, g, g), axis=0).transpose(2, 1, 3, 0)
    return jnp.asarray(g3.reshape(-1, 3), jnp.float32)


def _trilinear_border(vol, pts):
    D1, D2, D3 = vol.shape

    def pix(c, size):
        return jnp.clip((c + 1.0) * 0.5 * (size - 1), 0.0, size - 1.0)

    px = pix(pts[:, 0], D3)
    py = pix(pts[:, 1], D2)
    pz = pix(pts[:, 2], D1)
    x0f, y0f, z0f = jnp.floor(px), jnp.floor(py), jnp.floor(pz)
    wx, wy, wz = px - x0f, py - y0f, pz - z0f

    def ii(fv, size):
        return jnp.clip(fv, 0, size - 1).astype(jnp.int32)

    x0, x1 = ii(x0f, D3), ii(x0f + 1, D3)
    y0, y1 = ii(y0f, D2), ii(y0f + 1, D2)
    z0, z1 = ii(z0f, D1), ii(z0f + 1, D1)

    def g(zi, yi, xi):
        return vol[zi, yi, xi]

    c00 = g(z0, y0, x0) * (1 - wx) + g(z0, y0, x1) * wx
    c01 = g(z0, y1, x0) * (1 - wx) + g(z0, y1, x1) * wx
    c10 = g(z1, y0, x0) * (1 - wx) + g(z1, y0, x1) * wx
    c11 = g(z1, y1, x0) * (1 - wx) + g(z1, y1, x1) * wx
    c0 = c00 * (1 - wy) + c01 * wy
    c1 = c10 * (1 - wy) + c11 * wy
    return c0 * (1 - wz) + c1 * wz


def _scatter_kernel(i0_ref, i1_ref, i2_ref, nf_ref, o_ref, *, bs, nbi):
    g = pl.program_id(0)
    nb = pl.program_id(1)

    @pl.when(nb == 0)
    def _():
        o_ref[...] = jnp.zeros_like(o_ref)

    base = (g * nbi + nb) * bs

    def body(j, carry):
        a = i0_ref[base + j]
        b = i1_ref[base + j]
        c = i2_ref[base + j]
        sa = o_ref[pl.ds(a, 1), 0, :] + nf_ref[pl.ds(3 * j, 1), 0, :]
        sb = o_ref[pl.ds(b, 1), 0, :] + nf_ref[pl.ds(3 * j + 1, 1), 0, :]
        sc = o_ref[pl.ds(c, 1), 0, :] + nf_ref[pl.ds(3 * j + 2, 1), 0, :]
        o_ref[pl.ds(a, 1), 0, :] = sa
        o_ref[pl.ds(b, 1), 0, :] = sb
        o_ref[pl.ds(c, 1), 0, :] = sc
        return carry

    jax.lax.fori_loop(0, bs, body, 0)


def _vertex_normals(v, faces, m):
    i0, i1, i2 = faces[:, 0], faces[:, 1], faces[:, 2]
    nf = jnp.cross(v[i1] - v[i0], v[i2] - v[i0])
    F = nf.shape[0]
    c10 = i1 == i0
    c20 = i2 == i0
    c21 = i2 == i1
    f1 = lambda c: c.astype(jnp.float32)
    w0 = 1.0 + f1(c10) + f1(c20)
    w1 = (1.0 - f1(c10)) * (1.0 + f1(c21) * (1.0 - f1(c20)))
    w2 = f1(~(c20 | c21))
    i1 = jnp.where(c10, m, i1)
    i2 = jnp.where(c20 | c21, m + 1, i2)
    ups = jnp.stack([nf * w0[:, None], nf * w1[:, None], nf * w2[:, None]],
                    axis=1).reshape(3 * F, 1, 3)
    BS = 4096
    NBI = (F // 2) // BS
    out = pl.pallas_call(
        functools.partial(_scatter_kernel, bs=BS, nbi=NBI),
        out_shape=jax.ShapeDtypeStruct((2 * (m + 8), 1, 3), jnp.float32),
        grid_spec=pltpu.PrefetchScalarGridSpec(
            num_scalar_prefetch=3,
            grid=(2, NBI),
            in_specs=[pl.BlockSpec((3 * BS, 1, 3),
                                   lambda g, nb, *pf: (g * NBI + nb, 0, 0))],
            out_specs=pl.BlockSpec((m + 8, 1, 3), lambda g, nb, *pf: (g, 0, 0)),
        ),
        compiler_params=pltpu.CompilerParams(
            dimension_semantics=("parallel", "arbitrary"),
            disable_bounds_checks=True,
            vmem_limit_bytes=48 * 1024 * 1024),
    )(i0, i1, i2, ups)
    n = out.reshape(2, m + 8, 3)[0, :m] + out.reshape(2, m + 8, 3)[1, :m]
    return n / jnp.maximum(jnp.linalg.norm(n, axis=1, keepdims=True), 1e-6)


def _pad_cols(a, Np, dtype=jnp.bfloat16):
    m, n = a.shape
    return jnp.zeros((m, Np), dtype).at[:, :n].set(a.astype(dtype))


def kernel(x, V, f, blk_rows, blk_cols, blk_firsts, blk_lasts, A_blocks,
           wlc, blc, wlf, blf, w1, b1, w2a, w2b, b2, w3, b3,
           gcn_w0, gcn_b0, gcn_w1, gcn_b1, gcn_w2, gcn_b2,
           gcn_w3, gcn_b3, gcn_w4, gcn_b4):
    del blk_firsts, blk_lasts
    K, sf, ncls = 5, 0.1, 10
    m = x.shape[1]
    vol = V[0, 0]
    D1, D2, D3 = vol.shape
    D = max(D1, D2, D3)
    v = x[0]

    shift = _cube_shift(K) * (2.0 / D)
    rescale = jnp.asarray([D3 / D, D2 / D, D1 / D], jnp.float32)
    del rescale, shift
    doff = np.linspace(-K // 2, K // 2, K) * (2.0 / D) * 0.5 * (D - 1)

    def base_of(c):
        pix = (c + 1.0) * 0.5 * (D - 1)
        smin = jnp.clip(pix + doff[0], 0.0, D - 1.0)
        return jnp.clip(jnp.floor(smin), 0, D - 7).astype(jnp.int32), pix

    bx, px = base_of(v[:, 0])
    by, py = base_of(v[:, 1])
    bz, pz = base_of(v[:, 2])
    ii = np.arange(7, dtype=np.int32)
    off3 = ((ii[:, None, None] * D + ii[None, :, None]) * D
            + ii[None, None, :])
    fbase = (bz * D + by) * D + bx
    idx = jnp.asarray(off3)[..., None] + fbase[None, None, None, :]
    g7 = vol.reshape(-1).at[idx].get(
        mode="promise_in_bounds")
    meta = jnp.stack([px, py, pz, bx.astype(jnp.float32),
                      by.astype(jnp.float32), bz.astype(jnp.float32),
                      jnp.zeros_like(px), jnp.zeros_like(px)])
    nb_p = _interp_cubes(g7, meta, doff, D)

    fc = f[0]
    nfc = jnp.cross(v[fc[:, 1]] - v[fc[:, 0]], v[fc[:, 2]] - v[fc[:, 0]])
    nsum = jnp.zeros_like(v).at[fc.T.reshape(-1)].add(
        jnp.broadcast_to(nfc, (3,) + nfc.shape).reshape(-1, 3),
        mode="promise_in_bounds")
    normal = nsum / jnp.maximum(
        jnp.linalg.norm(nsum, axis=1, keepdims=True), 1e-6)
    x6_p = _pad_cols(jnp.concatenate([v, normal], axis=1), 128)
    z = _mlp_chain(nb_p, x6_p, wlc, blc, wlf, blf,
                   w1, b1, w2a, w2b, b2, w3, b3)

    nbk = m // _BLK
    rb = jnp.arange(nbk, dtype=jnp.int32)
    seg_start = jnp.searchsorted(blk_rows, rb, side="left").astype(jnp.int32)
    seg_end = jnp.searchsorted(blk_rows, rb, side="right").astype(jnp.int32)
    seg_cnt = seg_end - seg_start

    h = z
    for (wg, bg) in ((gcn_w0, gcn_b0), (gcn_w1, gcn_b1),
                     (gcn_w2, gcn_b2), (gcn_w3, gcn_b3)):
        h = _gcn_banded(seg_start, seg_cnt, blk_cols, A_blocks, h, wg, bg,
                        slope=0.2, out_dtype=jnp.bfloat16)
    out = _gcn_banded(seg_start, seg_cnt, blk_cols, A_blocks, h,
                      gcn_w4, gcn_b4, final_cfg=(sf, ncls),
                      out_dtype=jnp.float32)

    dx = out[:m, :3]
    logits = out[:m, 3:3 + ncls]
    return dx[None], logits
